# Initial kernel scaffold; baseline (speedup 1.0000x reference)
import jax
import jax.numpy as jnp
from jax.experimental import pallas as pl


def kernel(input_ids, W_embed, W_proj, b_proj):
    raise NotImplementedError("write your pallas kernel here")


# speedup vs baseline: 12.6334x; 12.6334x over previous
import functools

import jax
import jax.numpy as jnp
from jax import lax
from jax.experimental import pallas as pl
from jax.experimental.pallas import tpu as pltpu
from jax.experimental.pallas import tpu_sc as plsc

VOCAB = 64
EMBED = 16
B = 4
T = 8192
N = B * T
NPAIR2 = N // 2
PAIRS = B * (T - 1)

NC = 2
NS = 16
NW = NC * NS
PROW_W = NPAIR2 // NW
CHUNK = 128
NCHUNK = PROW_W // CHUNK

LOSS_W = NS
PAIRS_PER_W = N // LOSS_W
LCHUNK = PAIRS_PER_W // CHUNK
LANES = 16


def _tables_body(we_ref, wp_ref, b_ref, pair_ref, nll_ref):
    l_tab = (
        jnp.dot(we_ref[...], wp_ref[...], preferred_element_type=jnp.float32)
        + b_ref[...]
    )
    m = jnp.max(l_tab, axis=1, keepdims=True)
    lse = jnp.log(jnp.sum(jnp.exp(l_tab - m), axis=1, keepdims=True)) + m
    nll_ref[...] = lse - l_tab
    left = jnp.broadcast_to(l_tab[:, None, :], (VOCAB, VOCAB, VOCAB))
    right = jnp.broadcast_to(l_tab[None, :, :], (VOCAB, VOCAB, VOCAB))
    pair_ref[...] = jnp.concatenate(
        [left.reshape(VOCAB * VOCAB, VOCAB), right.reshape(VOCAB * VOCAB, VOCAB)],
        axis=1,
    )


_tables = pl.pallas_call(
    _tables_body,
    out_shape=[
        jax.ShapeDtypeStruct((VOCAB * VOCAB, 2 * VOCAB), jnp.float32),
        jax.ShapeDtypeStruct((VOCAB, VOCAB), jnp.float32),
    ],
)


_mesh = plsc.VectorSubcoreMesh(core_axis_name="c", subcore_axis_name="s")


@functools.partial(
    pl.kernel,
    mesh=_mesh,
    out_type=[
        jax.ShapeDtypeStruct((NPAIR2, 2 * VOCAB), jnp.float32),
        jax.ShapeDtypeStruct((LANES,), jnp.float32),
    ],
    scratch_types=[
        pltpu.VMEM((NCHUNK, CHUNK), jnp.int32),
        pltpu.VMEM((PROW_W, 2 * VOCAB), jnp.float32),
        pltpu.VMEM((PAIRS_PER_W,), jnp.int32),
        pltpu.VMEM((PAIRS_PER_W,), jnp.float32),
        pltpu.VMEM((LANES,), jnp.float32),
        pltpu.VMEM((LOSS_W * LANES,), jnp.float32),
        pltpu.VMEM_SHARED((LOSS_W * LANES,), jnp.float32),
        pltpu.SemaphoreType.DMA,
        pltpu.SemaphoreType.DMA,
    ],
)
def _sc_gather(pidx_hbm, lidx_hbm, pair_hbm, nll_hbm, out_hbm, loss_hbm,
               idx_v, rows_v, p_v, vals_v, stage_v, part_v, part_sh,
               sem, sem2):
    cid = lax.axis_index("c")
    sid = lax.axis_index("s")
    wid = sid * NC + cid
    base = wid * PROW_W

    pltpu.sync_copy(pidx_hbm.at[pl.ds(wid * NCHUNK, NCHUNK)], idx_v)
    handles = []
    for j in range(NCHUNK):
        handles.append(
            pltpu.async_copy(
                pair_hbm.at[idx_v.at[j]],
                rows_v.at[pl.ds(j * CHUNK, CHUNK)],
                sem,
            )
        )
    for h in handles:
        h.wait()
    pltpu.sync_copy(rows_v, out_hbm.at[pl.ds(base, PROW_W)])

    @pl.when(cid == 0)
    def _loss_partial():
        lbase = sid * PAIRS_PER_W
        pltpu.sync_copy(lidx_hbm.at[pl.ds(lbase, PAIRS_PER_W)], p_v)
        lhandles = []
        for j in range(LCHUNK):
            lhandles.append(
                pltpu.async_copy(
                    nll_hbm.at[p_v.at[pl.ds(j * CHUNK, CHUNK)]],
                    vals_v.at[pl.ds(j * CHUNK, CHUNK)],
                    sem2,
                )
            )
        for h in lhandles:
            h.wait()

        def abody(i, acc):
            gidx = lbase + i * LANES + lax.iota(jnp.int32, LANES)
            vals = vals_v[pl.ds(i * LANES, LANES)]
            return acc + jnp.where(gidx < PAIRS, vals, 0.0)

        acc = lax.fori_loop(
            0, PAIRS_PER_W // LANES, abody, jnp.zeros((LANES,), jnp.float32)
        )
        stage_v[...] = acc
        pltpu.sync_copy(stage_v, part_sh.at[pl.ds(sid * LANES, LANES)])

    plsc.subcore_barrier()

    @pl.when(jnp.logical_and(cid == 0, sid == 0))
    def _loss_final():
        pltpu.sync_copy(part_sh, part_v)

        def body(i, acc):
            return acc + part_v[pl.ds(i * LANES, LANES)]

        tot = lax.fori_loop(
            0, LOSS_W, body, jnp.zeros((LANES,), jnp.float32)
        )
        total = tot[0]
        for i in range(1, LANES):
            total = total + tot[i]
        mean = total * (1.0 / PAIRS)
        stage_v[...] = jnp.zeros((LANES,), jnp.float32) + mean
        pltpu.sync_copy(stage_v, loss_hbm)


def kernel(input_ids, W_embed, W_proj, b_proj):
    ids = input_ids.astype(jnp.int32)
    pair_tab, nll_tab = _tables(W_embed, W_proj, b_proj.reshape(1, VOCAB))

    ids_flat = ids.reshape(-1)
    even = ids_flat[0::2]
    odd = ids_flat[1::2]
    pidx = (even * VOCAB + odd).reshape(NW * NCHUNK, CHUNK)

    pad = jnp.zeros((N - PAIRS,), jnp.int32)
    c_pad = jnp.concatenate([ids[:, :-1].reshape(-1), pad])
    n_pad = jnp.concatenate([ids[:, 1:].reshape(-1), pad])
    lidx = c_pad * VOCAB + n_pad

    logits_rows, loss_vec = _sc_gather(pidx, lidx, pair_tab, nll_tab.reshape(-1))
    return loss_vec[0], logits_rows.reshape(B, T, VOCAB)
